# Initial kernel scaffold; baseline (speedup 1.0000x reference)
import jax
import jax.numpy as jnp
from jax.experimental import pallas as pl


def kernel(indices, table):
    raise NotImplementedError("write your pallas kernel here")


# speedup vs baseline: 6.1583x; 6.1583x over previous
import functools

import jax
import jax.numpy as jnp
from jax import lax
from jax.experimental import pallas as pl
from jax.experimental.pallas import tpu as pltpu
from jax.experimental.pallas import tpu_sc as plsc

NC = 2
NS = 16
NW = NC * NS
CHUNK = 128


@functools.partial(jax.jit, static_argnums=(2, 3))
def _gather_rows(table, idx, n_chunks, d):
    mesh = plsc.VectorSubcoreMesh(core_axis_name="c", subcore_axis_name="s")
    n_per_w = n_chunks * CHUNK

    @functools.partial(
        pl.kernel,
        mesh=mesh,
        out_type=jax.ShapeDtypeStruct((NW * n_per_w, d), jnp.float32),
        scratch_types=[
            pltpu.VMEM((n_chunks, CHUNK), jnp.int32),
            pltpu.VMEM((CHUNK, d), jnp.float32),
            pltpu.SemaphoreType.DMA,
        ],
        compiler_params=pltpu.CompilerParams(use_tc_tiling_on_sc=False),
    )
    def k(table_hbm, idx_hbm, out_hbm, idx_v, rows_v, gsem):
        wid = lax.axis_index("s") * NC + lax.axis_index("c")
        base = wid * n_per_w
        pltpu.sync_copy(idx_hbm.at[wid], idx_v)

        def body(j, carry):
            pltpu.async_copy(table_hbm.at[idx_v.at[j]], rows_v, gsem).wait()
            pltpu.sync_copy(rows_v, out_hbm.at[pl.ds(base + j * CHUNK, CHUNK)])
            return carry

        lax.fori_loop(0, n_chunks, body, 0)

    return k(table, idx)


def kernel(indices, table):
    b, s, p = indices.shape
    d = table.shape[1]
    total = b * s * p
    n_chunks = total // (NW * CHUNK)
    idx = indices.reshape(NW, n_chunks, CHUNK).astype(jnp.int32)
    out = _gather_rows(table, idx, n_chunks, d)
    return out.reshape(b, s, p * d)


# speedup vs baseline: 6.5034x; 1.0560x over previous
import functools

import jax
import jax.numpy as jnp
from jax import lax
from jax.experimental import pallas as pl
from jax.experimental.pallas import tpu as pltpu
from jax.experimental.pallas import tpu_sc as plsc

NC = 2
NS = 16
NW = NC * NS
CHUNK = 128
K = 5


@functools.partial(jax.jit, static_argnums=(2, 3))
def _gather_rows(table, idx, n_chunks, d):
    mesh = plsc.VectorSubcoreMesh(core_axis_name="c", subcore_axis_name="s")
    n_super = n_chunks // K
    assert n_chunks % K == 0 and n_super % 2 == 0

    @functools.partial(
        pl.kernel,
        mesh=mesh,
        out_type=jax.ShapeDtypeStruct((NW * n_super, K, CHUNK, d),
                                      jnp.float32),
        scratch_types=[
            pltpu.VMEM((n_chunks, CHUNK), jnp.int32),
            pltpu.VMEM((K, CHUNK, d), jnp.float32),
            pltpu.VMEM((K, CHUNK, d), jnp.float32),
            pltpu.SemaphoreType.DMA,
            pltpu.SemaphoreType.DMA,
            pltpu.SemaphoreType.DMA,
        ],
        compiler_params=pltpu.CompilerParams(use_tc_tiling_on_sc=False),
    )
    def k(table_hbm, idx_hbm, out_hbm, idx_v, buf0, buf1, gsem, wsem0, wsem1):
        wid = lax.axis_index("s") * NC + lax.axis_index("c")
        base = wid * n_super
        pltpu.sync_copy(idx_hbm.at[wid], idx_v)

        def super_chunk(sc, buf, wsem, first):
            @pl.when(jnp.logical_not(first))
            def _():
                pltpu.make_async_copy(buf, out_hbm.at[base], wsem).wait()
            cps = []
            for t in range(K):
                cps.append(pltpu.async_copy(
                    table_hbm.at[idx_v.at[sc * K + t]], buf.at[t], gsem))
            for cp in cps:
                cp.wait()
            pltpu.async_copy(buf, out_hbm.at[base + sc], wsem)

        def body(p, carry):
            super_chunk(2 * p, buf0, wsem0, p == 0)
            super_chunk(2 * p + 1, buf1, wsem1, p == 0)
            return carry

        lax.fori_loop(0, n_super // 2, body, 0)
        pltpu.make_async_copy(buf0, out_hbm.at[base], wsem0).wait()
        pltpu.make_async_copy(buf1, out_hbm.at[base], wsem1).wait()

    return k(table, idx)


def kernel(indices, table):
    b, s, p = indices.shape
    d = table.shape[1]
    total = b * s * p
    n_chunks = total // (NW * CHUNK)
    idx = indices.reshape(NW, n_chunks, CHUNK).astype(jnp.int32)
    out = _gather_rows(table, idx, n_chunks, d)
    return out.reshape(b, s, p * d)


# speedup vs baseline: 16.3496x; 2.5140x over previous
import functools

import jax
import jax.numpy as jnp
from jax import lax
from jax.experimental import pallas as pl
from jax.experimental.pallas import tpu as pltpu
from jax.experimental.pallas import tpu_sc as plsc

NC = 2
NS = 16
NW = NC * NS
CHUNK = 128
K = 5


@functools.partial(jax.jit, static_argnums=(2, 3))
def _gather_rows(table, idx, n_chunks, d):
    mesh = plsc.VectorSubcoreMesh(core_axis_name="c", subcore_axis_name="s")
    n_super = n_chunks // K
    assert n_chunks % K == 0 and n_super % 2 == 0

    @functools.partial(
        pl.kernel,
        mesh=mesh,
        out_type=jax.ShapeDtypeStruct((NW * n_super, K, CHUNK, d),
                                      jnp.float32),
        scratch_types=[
            pltpu.VMEM((n_chunks, CHUNK), jnp.int32),
            pltpu.VMEM((K, CHUNK, d), jnp.float32),
            pltpu.VMEM((K, CHUNK, d), jnp.float32),
            pltpu.VMEM_SHARED((103, d), jnp.float32),
            pltpu.SemaphoreType.DMA,
            pltpu.SemaphoreType.DMA,
            pltpu.SemaphoreType.DMA,
        ],
        compiler_params=pltpu.CompilerParams(use_tc_tiling_on_sc=False),
    )
    def k(table_hbm, idx_hbm, out_hbm, idx_v, buf0, buf1, table_v, gsem,
          wsem0, wsem1):
        wid = lax.axis_index("s") * NC + lax.axis_index("c")
        base = wid * n_super
        @pl.when(lax.axis_index("s") == 0)
        def _():
            pltpu.sync_copy(table_hbm, table_v)
        pltpu.sync_copy(idx_hbm.at[wid], idx_v)
        plsc.subcore_barrier()

        def super_chunk(sc, buf, wsem, first):
            @pl.when(jnp.logical_not(first))
            def _():
                pltpu.make_async_copy(buf, out_hbm.at[base], wsem).wait()
            cps = []
            for t in range(K):
                cps.append(pltpu.async_copy(
                    table_v.at[idx_v.at[sc * K + t]], buf.at[t], gsem))
            for cp in cps:
                cp.wait()
            pltpu.async_copy(buf, out_hbm.at[base + sc], wsem)

        def body(p, carry):
            super_chunk(2 * p, buf0, wsem0, p == 0)
            super_chunk(2 * p + 1, buf1, wsem1, p == 0)
            return carry

        lax.fori_loop(0, n_super // 2, body, 0)
        pltpu.make_async_copy(buf0, out_hbm.at[base], wsem0).wait()
        pltpu.make_async_copy(buf1, out_hbm.at[base], wsem1).wait()

    return k(table, idx)


def kernel(indices, table):
    b, s, p = indices.shape
    d = table.shape[1]
    total = b * s * p
    n_chunks = total // (NW * CHUNK)
    idx = indices.reshape(NW, n_chunks, CHUNK).astype(jnp.int32)
    out = _gather_rows(table, idx, n_chunks, d)
    return out.reshape(b, s, p * d)
